# Initial kernel scaffold; baseline (speedup 1.0000x reference)
import jax
import jax.numpy as jnp
from jax.experimental import pallas as pl


def kernel(hidden_states, router_w, gate_up_proj, down_proj, gate_w, up_w, down_w):
    raise NotImplementedError("write your pallas kernel here")


# speedup vs baseline: 1.0740x; 1.0740x over previous
import functools

import jax
import jax.numpy as jnp
from jax import lax
from jax.experimental import pallas as pl
from jax.experimental.pallas import tpu as pltpu
from jax.experimental.pallas import tpu_sc as plsc

E = 8
H = 768
F = 1024
T = 2048
B = 128
NB = T // B

NC = 2
NS = 16
NW = NC * NS
BPW = T // NW


def _router_body(hs_ref, rw_ref, pos_ref, score_ref, base_ref):
    hs = hs_ref[...]
    logits = jnp.dot(hs, rw_ref[...].T, preferred_element_type=jnp.float32)
    m = jnp.max(logits, axis=1, keepdims=True)
    score_ref[...] = jax.nn.sigmoid(m)
    idx8 = lax.broadcasted_iota(jnp.int32, (T, E), 1)
    eid = jnp.min(jnp.where(logits == m, idx8, E), axis=1, keepdims=True)
    oh = (idx8 == eid).astype(jnp.float32)
    tri = (lax.broadcasted_iota(jnp.int32, (B, B), 0)
           >= lax.broadcasted_iota(jnp.int32, (B, B), 1)).astype(jnp.float32)

    run = jnp.zeros((1, E), jnp.float32)
    ranks = []
    for c in range(NB):
        ohc = oh[c * B:(c + 1) * B, :]
        csum = jnp.dot(tri, ohc, preferred_element_type=jnp.float32,
                       precision=lax.Precision.HIGHEST)
        ranks.append(jnp.sum(ohc * csum, axis=1, keepdims=True) - 1.0
                     + jnp.sum(ohc * run, axis=1, keepdims=True))
        run = run + csum[B - 1:B, :]
    triE = (lax.broadcasted_iota(jnp.int32, (E, E), 0)
            < lax.broadcasted_iota(jnp.int32, (E, E), 1)).astype(jnp.float32)
    base = jnp.dot(run, triE, preferred_element_type=jnp.float32,
                   precision=lax.Precision.HIGHEST)
    base_ref[...] = base.astype(jnp.int32)
    for c in range(NB):
        ohc = oh[c * B:(c + 1) * B, :]
        badd = jnp.sum(ohc * base, axis=1, keepdims=True)
        pos_ref[c * B:(c + 1) * B, :] = (ranks[c] + badd).astype(jnp.int32)


def _router(hs, router_w):
    return pl.pallas_call(
        _router_body,
        out_shape=(
            jax.ShapeDtypeStruct((T, 1), jnp.int32),
            jax.ShapeDtypeStruct((T, 1), jnp.float32),
            jax.ShapeDtypeStruct((1, E), jnp.int32),
        ),
    )(hs, router_w)


def _disperse_body(hs_hbm, score_hbm, pos_hbm, xs_hbm, ss_hbm,
                   idx_v, rows_v, sv_v, sem_r, sem_s):
    wid = lax.axis_index("s") * NC + lax.axis_index("c")
    base = wid * BPW
    pltpu.sync_copy(pos_hbm.at[pl.ds(base, BPW)], idx_v)
    pltpu.sync_copy(hs_hbm.at[pl.ds(base, BPW)], rows_v)
    pltpu.sync_copy(score_hbm.at[pl.ds(base, BPW)], sv_v)
    pltpu.async_copy(rows_v, xs_hbm.at[idx_v], sem_r).wait()
    pltpu.async_copy(sv_v, ss_hbm.at[idx_v], sem_s).wait()


def _disperse(hs, score, pos):
    mesh = plsc.VectorSubcoreMesh(core_axis_name="c", subcore_axis_name="s",
                                  num_cores=NC, num_subcores=NS)
    return pl.kernel(
        _disperse_body,
        out_type=(
            jax.ShapeDtypeStruct((T, H), jnp.float32),
            jax.ShapeDtypeStruct((T,), jnp.float32),
        ),
        mesh=mesh,
        scratch_types=[
            pltpu.VMEM((BPW,), jnp.int32),
            pltpu.VMEM((BPW, H), jnp.float32),
            pltpu.VMEM((BPW,), jnp.float32),
            pltpu.SemaphoreType.DMA,
            pltpu.SemaphoreType.DMA,
        ],
    )(hs, score, pos)


def _moe_body(off_ref, xs_ref, hs_ref, ss_ref, wgu_ref, wd_ref,
              wg_ref, wu_ref, wdw_ref, routed_ref, shared_ref):
    e = pl.program_id(0)
    b = pl.program_id(1)
    r0 = b * B
    start = off_ref[e]
    end = off_ref[e + 1]

    @pl.when(e == E)
    def _shared():
        x = hs_ref[pl.ds(r0, B), :]
        dn = (((1,), (1,)), ((), ()))
        g = lax.dot_general(x, wg_ref[...], dn,
                            preferred_element_type=jnp.float32)
        u = lax.dot_general(x, wu_ref[...], dn,
                            preferred_element_type=jnp.float32)
        act = u * (g * jax.nn.sigmoid(g))
        shared_ref[pl.ds(r0, B), :] = lax.dot_general(
            act, wdw_ref[...], dn, preferred_element_type=jnp.float32)

    @pl.when(jnp.logical_and(e < E,
                             jnp.logical_and(r0 < end, r0 + B > start)))
    def _routed():
        x = xs_ref[pl.ds(r0, B), :] * ss_ref[pl.ds(r0, B), :]
        gu = jnp.dot(x, wgu_ref[0], preferred_element_type=jnp.float32)
        g = gu[:, :F]
        u = gu[:, F:]
        act = u * (g * jax.nn.sigmoid(g))
        r = jnp.dot(act, wd_ref[0], preferred_element_type=jnp.float32)
        rows = r0 + lax.broadcasted_iota(jnp.int32, (B, 1), 0)
        keep = jnp.logical_and(rows >= start, rows < end)
        routed_ref[pl.ds(r0, B), :] = jnp.where(
            keep, r, routed_ref[pl.ds(r0, B), :])


def _moe(off, xs, hs, ss, gate_up_proj, down_proj, gate_w, up_w, down_w):
    return pl.pallas_call(
        _moe_body,
        grid=(E + 1, NB),
        in_specs=[
            pl.BlockSpec(memory_space=pltpu.SMEM),
            pl.BlockSpec((T, H), lambda e, b: (0, 0)),
            pl.BlockSpec((T, H), lambda e, b: (0, 0)),
            pl.BlockSpec((T, 1), lambda e, b: (0, 0)),
            pl.BlockSpec((1, H, 2 * F), lambda e, b: (jnp.minimum(e, E - 1), 0, 0)),
            pl.BlockSpec((1, F, H), lambda e, b: (jnp.minimum(e, E - 1), 0, 0)),
            pl.BlockSpec((F, H), lambda e, b: (0, 0)),
            pl.BlockSpec((F, H), lambda e, b: (0, 0)),
            pl.BlockSpec((H, F), lambda e, b: (0, 0)),
        ],
        out_specs=(
            pl.BlockSpec((T, H), lambda e, b: (0, 0)),
            pl.BlockSpec((T, H), lambda e, b: (0, 0)),
        ),
        out_shape=(
            jax.ShapeDtypeStruct((T, H), jnp.float32),
            jax.ShapeDtypeStruct((T, H), jnp.float32),
        ),
        compiler_params=pltpu.CompilerParams(
            dimension_semantics=("arbitrary", "arbitrary")),
    )(off, xs, hs, ss, gate_up_proj, down_proj, gate_w, up_w, down_w)


def _combine_body(shared_hbm, routed_hbm, pos_hbm, out_hbm,
                  idx_v, acc_v, rows_v, sem):
    wid = lax.axis_index("s") * NC + lax.axis_index("c")
    base = wid * BPW
    pltpu.sync_copy(pos_hbm.at[pl.ds(base, BPW)], idx_v)
    pltpu.sync_copy(shared_hbm.at[pl.ds(base, BPW)], acc_v)
    pltpu.async_copy(routed_hbm.at[idx_v], rows_v, sem).wait()

    def row_add(r, carry):
        for c in range(H // 16):
            sl = pl.ds(c * 16, 16)
            acc_v[r, sl] = acc_v[r, sl] + rows_v[r, sl]
        return carry

    lax.fori_loop(0, BPW, row_add, 0)
    pltpu.sync_copy(acc_v, out_hbm.at[pl.ds(base, BPW)])


def _combine(shared, routed, pos):
    mesh = plsc.VectorSubcoreMesh(core_axis_name="c", subcore_axis_name="s",
                                  num_cores=NC, num_subcores=NS)
    return pl.kernel(
        _combine_body,
        out_type=jax.ShapeDtypeStruct((T, H), jnp.float32),
        mesh=mesh,
        scratch_types=[
            pltpu.VMEM((BPW,), jnp.int32),
            pltpu.VMEM((BPW, H), jnp.float32),
            pltpu.VMEM((BPW, H), jnp.float32),
            pltpu.SemaphoreType.DMA,
        ],
    )(shared, routed, pos)


def kernel(hidden_states, router_w, gate_up_proj, down_proj,
           gate_w, up_w, down_w):
    orig_shape = hidden_states.shape
    hs = hidden_states.reshape(-1, H)
    pos2d, score2d, base2d = _router(hs, router_w)
    pos = pos2d.reshape(T)
    score = score2d.reshape(T)
    off = jnp.concatenate(
        [base2d.reshape(E), jnp.full((E,), T, jnp.int32)])
    xs, s_sorted = _disperse(hs, score, pos)
    routed, shared = _moe(off, xs, hs, s_sorted.reshape(T, 1),
                          gate_up_proj, down_proj, gate_w, up_w, down_w)
    out = _combine(shared, routed, pos)
    return out.reshape(orig_shape)


# speedup vs baseline: 1.1480x; 1.0690x over previous
import functools

import jax
import jax.numpy as jnp
from jax import lax
from jax.experimental import pallas as pl
from jax.experimental.pallas import tpu as pltpu
from jax.experimental.pallas import tpu_sc as plsc

E = 8
H = 768
F = 1024
T = 2048
B = 128
NB = T // B

NC = 2
NS = 16
NW = NC * NS
BPW = T // NW


def _router_body(hs_ref, rw_ref, pos_ref, score_ref, base_ref):
    hs = hs_ref[...]
    logits = jnp.dot(hs, rw_ref[...].T, preferred_element_type=jnp.float32)
    m = jnp.max(logits, axis=1, keepdims=True)
    score_ref[...] = jax.nn.sigmoid(m)
    idx8 = lax.broadcasted_iota(jnp.int32, (T, E), 1)
    eid = jnp.min(jnp.where(logits == m, idx8, E), axis=1, keepdims=True)
    oh = (idx8 == eid).astype(jnp.float32)
    tri = (lax.broadcasted_iota(jnp.int32, (B, B), 0)
           >= lax.broadcasted_iota(jnp.int32, (B, B), 1)).astype(jnp.float32)

    run = jnp.zeros((1, E), jnp.float32)
    ranks = []
    for c in range(NB):
        ohc = oh[c * B:(c + 1) * B, :]
        csum = jnp.dot(tri, ohc, preferred_element_type=jnp.float32,
                       precision=lax.Precision.HIGHEST)
        ranks.append(jnp.sum(ohc * csum, axis=1, keepdims=True) - 1.0
                     + jnp.sum(ohc * run, axis=1, keepdims=True))
        run = run + csum[B - 1:B, :]
    triE = (lax.broadcasted_iota(jnp.int32, (E, E), 0)
            < lax.broadcasted_iota(jnp.int32, (E, E), 1)).astype(jnp.float32)
    base = jnp.dot(run, triE, preferred_element_type=jnp.float32,
                   precision=lax.Precision.HIGHEST)
    base_ref[...] = base.astype(jnp.int32)
    for c in range(NB):
        ohc = oh[c * B:(c + 1) * B, :]
        badd = jnp.sum(ohc * base, axis=1, keepdims=True)
        pos_ref[c * B:(c + 1) * B, :] = (ranks[c] + badd).astype(jnp.int32)


def _router(hs, router_w):
    return pl.pallas_call(
        _router_body,
        out_shape=(
            jax.ShapeDtypeStruct((T, 1), jnp.int32),
            jax.ShapeDtypeStruct((T, 1), jnp.float32),
            jax.ShapeDtypeStruct((1, E), jnp.int32),
        ),
    )(hs, router_w)


def _disperse_body(hs_hbm, score_hbm, pos_hbm, xs_hbm, ss_hbm,
                   idx_v, rows_v, sv_v, sem_p, sem_r, sem_s, sem_r2, sem_s2):
    wid = lax.axis_index("s") * NC + lax.axis_index("c")
    base = wid * BPW
    cp_pos = pltpu.async_copy(pos_hbm.at[pl.ds(base, BPW)], idx_v, sem_p)
    cp_rows = pltpu.async_copy(hs_hbm.at[pl.ds(base, BPW)], rows_v, sem_r)
    cp_s = pltpu.async_copy(score_hbm.at[pl.ds(base, BPW)], sv_v, sem_s)
    cp_pos.wait()
    cp_rows.wait()
    sc_rows = pltpu.async_copy(rows_v, xs_hbm.at[idx_v], sem_r2)
    cp_s.wait()
    sc_s = pltpu.async_copy(sv_v, ss_hbm.at[idx_v], sem_s2)
    sc_rows.wait()
    sc_s.wait()


def _disperse(hs, score, pos):
    mesh = plsc.VectorSubcoreMesh(core_axis_name="c", subcore_axis_name="s",
                                  num_cores=NC, num_subcores=NS)
    return pl.kernel(
        _disperse_body,
        out_type=(
            jax.ShapeDtypeStruct((T, H), jnp.float32),
            jax.ShapeDtypeStruct((T,), jnp.float32),
        ),
        mesh=mesh,
        scratch_types=[
            pltpu.VMEM((BPW,), jnp.int32),
            pltpu.VMEM((BPW, H), jnp.float32),
            pltpu.VMEM((BPW,), jnp.float32),
            pltpu.SemaphoreType.DMA,
            pltpu.SemaphoreType.DMA,
            pltpu.SemaphoreType.DMA,
            pltpu.SemaphoreType.DMA,
            pltpu.SemaphoreType.DMA,
        ],
    )(hs, score, pos)


def _moe_body(off_ref, xs_ref, ss_ref, wgu_ref, wd_ref,
              wg_ref, wu_ref, wdw_ref, out_ref):
    e = pl.program_id(0)
    b = pl.program_id(1)
    r0 = b * B
    start = off_ref[e]
    end = off_ref[e + 1]
    dn = (((1,), (1,)), ((), ()))

    @pl.when(jnp.logical_and(e < E,
                             jnp.logical_and(r0 < end, r0 + B > start)))
    def _routed():
        x = xs_ref[pl.ds(r0, B), :] * ss_ref[pl.ds(r0, B), :]
        gu = jnp.dot(x, wgu_ref[0], preferred_element_type=jnp.float32)
        g = gu[:, :F]
        u = gu[:, F:]
        act = u * (g * jax.nn.sigmoid(g))
        r = jnp.dot(act, wd_ref[0], preferred_element_type=jnp.float32)
        rows = r0 + lax.broadcasted_iota(jnp.int32, (B, 1), 0)
        keep = jnp.logical_and(rows >= start, rows < end)
        out_ref[pl.ds(r0, B), :] = jnp.where(
            keep, r, out_ref[pl.ds(r0, B), :])

    @pl.when(e == E)
    def _shared():
        x = xs_ref[pl.ds(r0, B), :]
        g = lax.dot_general(x, wg_ref[...], dn,
                            preferred_element_type=jnp.float32)
        u = lax.dot_general(x, wu_ref[...], dn,
                            preferred_element_type=jnp.float32)
        act = u * (g * jax.nn.sigmoid(g))
        sh = lax.dot_general(act, wdw_ref[...], dn,
                             preferred_element_type=jnp.float32)
        out_ref[pl.ds(r0, B), :] = out_ref[pl.ds(r0, B), :] + sh


def _moe(off, xs, ss, gate_up_proj, down_proj, gate_w, up_w, down_w):
    return pl.pallas_call(
        _moe_body,
        grid=(E + 1, NB),
        in_specs=[
            pl.BlockSpec(memory_space=pltpu.SMEM),
            pl.BlockSpec((T, H), lambda e, b: (0, 0)),
            pl.BlockSpec((T, 1), lambda e, b: (0, 0)),
            pl.BlockSpec((1, H, 2 * F), lambda e, b: (jnp.minimum(e, E - 1), 0, 0)),
            pl.BlockSpec((1, F, H), lambda e, b: (jnp.minimum(e, E - 1), 0, 0)),
            pl.BlockSpec((F, H), lambda e, b: (0, 0)),
            pl.BlockSpec((F, H), lambda e, b: (0, 0)),
            pl.BlockSpec((H, F), lambda e, b: (0, 0)),
        ],
        out_specs=pl.BlockSpec((T, H), lambda e, b: (0, 0)),
        out_shape=jax.ShapeDtypeStruct((T, H), jnp.float32),
        compiler_params=pltpu.CompilerParams(
            dimension_semantics=("arbitrary", "arbitrary")),
    )(off, xs, ss, gate_up_proj, down_proj, gate_w, up_w, down_w)


def _combine_body(osort_hbm, pos_hbm, out_hbm, idx_v, rows_v, sem):
    wid = lax.axis_index("s") * NC + lax.axis_index("c")
    base = wid * BPW
    pltpu.sync_copy(pos_hbm.at[pl.ds(base, BPW)], idx_v)
    pltpu.async_copy(osort_hbm.at[idx_v], rows_v, sem).wait()
    pltpu.sync_copy(rows_v, out_hbm.at[pl.ds(base, BPW)])


def _combine(osort, pos):
    mesh = plsc.VectorSubcoreMesh(core_axis_name="c", subcore_axis_name="s",
                                  num_cores=NC, num_subcores=NS)
    return pl.kernel(
        _combine_body,
        out_type=jax.ShapeDtypeStruct((T, H), jnp.float32),
        mesh=mesh,
        scratch_types=[
            pltpu.VMEM((BPW,), jnp.int32),
            pltpu.VMEM((BPW, H), jnp.float32),
            pltpu.SemaphoreType.DMA,
        ],
    )(osort, pos)


def kernel(hidden_states, router_w, gate_up_proj, down_proj,
           gate_w, up_w, down_w):
    orig_shape = hidden_states.shape
    hs = hidden_states.reshape(-1, H)
    pos2d, score2d, base2d = _router(hs, router_w)
    pos = pos2d.reshape(T)
    score = score2d.reshape(T)
    off = jnp.concatenate(
        [base2d.reshape(E), jnp.full((E,), T, jnp.int32)])
    xs, s_sorted = _disperse(hs, score, pos)
    osort = _moe(off, xs, s_sorted.reshape(T, 1),
                 gate_up_proj, down_proj, gate_w, up_w, down_w)
    out = _combine(osort, pos)
    return out.reshape(orig_shape)


# speedup vs baseline: 1.3111x; 1.1421x over previous
import functools

import jax
import jax.numpy as jnp
from jax import lax
from jax.experimental import pallas as pl
from jax.experimental.pallas import tpu as pltpu
from jax.experimental.pallas import tpu_sc as plsc

E = 8
H = 768
F = 1024
T = 2048
B = 256
NB = T // B

NC = 2
NS = 16
NW = NC * NS
BPW = T // NW


def _router_body(hs_ref, rw_ref, pos_ref, score_ref, base_ref):
    hs = hs_ref[...]
    logits = jnp.dot(hs, rw_ref[...].T, preferred_element_type=jnp.float32)
    m = jnp.max(logits, axis=1, keepdims=True)
    score_ref[...] = jax.nn.sigmoid(m)
    idx8 = lax.broadcasted_iota(jnp.int32, (T, E), 1)
    eid = jnp.min(jnp.where(logits == m, idx8, E), axis=1, keepdims=True)
    oh = (idx8 == eid).astype(jnp.float32)
    tri = (lax.broadcasted_iota(jnp.int32, (B, B), 0)
           >= lax.broadcasted_iota(jnp.int32, (B, B), 1)).astype(jnp.float32)

    run = jnp.zeros((1, E), jnp.float32)
    ranks = []
    for c in range(NB):
        ohc = oh[c * B:(c + 1) * B, :]
        csum = jnp.dot(tri, ohc, preferred_element_type=jnp.float32,
                       precision=lax.Precision.HIGHEST)
        ranks.append(jnp.sum(ohc * csum, axis=1, keepdims=True) - 1.0
                     + jnp.sum(ohc * run, axis=1, keepdims=True))
        run = run + csum[B - 1:B, :]
    triE = (lax.broadcasted_iota(jnp.int32, (E, E), 0)
            < lax.broadcasted_iota(jnp.int32, (E, E), 1)).astype(jnp.float32)
    base = jnp.dot(run, triE, preferred_element_type=jnp.float32,
                   precision=lax.Precision.HIGHEST)
    base_ref[...] = base.astype(jnp.int32)
    for c in range(NB):
        ohc = oh[c * B:(c + 1) * B, :]
        badd = jnp.sum(ohc * base, axis=1, keepdims=True)
        pos_ref[c * B:(c + 1) * B, :] = (ranks[c] + badd).astype(jnp.int32)


def _router(hs, router_w):
    return pl.pallas_call(
        _router_body,
        out_shape=(
            jax.ShapeDtypeStruct((T, 1), jnp.int32),
            jax.ShapeDtypeStruct((T, 1), jnp.float32),
            jax.ShapeDtypeStruct((1, E), jnp.int32),
        ),
    )(hs, router_w)


def _disperse_body(hs_hbm, score_hbm, pos_hbm, xs_hbm, ss_hbm,
                   idx_v, rows_v, sv_v, sem_p, sem_r, sem_s, sem_r2, sem_s2):
    wid = lax.axis_index("s") * NC + lax.axis_index("c")
    base = wid * BPW
    cp_pos = pltpu.async_copy(pos_hbm.at[pl.ds(base, BPW)], idx_v, sem_p)
    cp_rows = pltpu.async_copy(hs_hbm.at[pl.ds(base, BPW)], rows_v, sem_r)
    cp_s = pltpu.async_copy(score_hbm.at[pl.ds(base, BPW)], sv_v, sem_s)
    cp_pos.wait()
    cp_rows.wait()
    sc_rows = pltpu.async_copy(rows_v, xs_hbm.at[idx_v], sem_r2)
    cp_s.wait()
    sc_s = pltpu.async_copy(sv_v, ss_hbm.at[idx_v], sem_s2)
    sc_rows.wait()
    sc_s.wait()


def _disperse(hs, score, pos):
    mesh = plsc.VectorSubcoreMesh(core_axis_name="c", subcore_axis_name="s",
                                  num_cores=NC, num_subcores=NS)
    return pl.kernel(
        _disperse_body,
        out_type=(
            jax.ShapeDtypeStruct((T, H), jnp.float32),
            jax.ShapeDtypeStruct((T,), jnp.float32),
        ),
        mesh=mesh,
        scratch_types=[
            pltpu.VMEM((BPW,), jnp.int32),
            pltpu.VMEM((BPW, H), jnp.float32),
            pltpu.VMEM((BPW,), jnp.float32),
            pltpu.SemaphoreType.DMA,
            pltpu.SemaphoreType.DMA,
            pltpu.SemaphoreType.DMA,
            pltpu.SemaphoreType.DMA,
            pltpu.SemaphoreType.DMA,
        ],
    )(hs, score, pos)


def _moe_body(off_ref, xs_ref, ss_ref, wgu_ref, wd_ref,
              wg_ref, wu_ref, wdw_ref, out_ref):
    e = pl.program_id(0)
    b = pl.program_id(1)
    r0 = b * B
    start = off_ref[e]
    end = off_ref[e + 1]
    dn = (((1,), (1,)), ((), ()))

    @pl.when(jnp.logical_and(e < E,
                             jnp.logical_and(r0 < end, r0 + B > start)))
    def _routed():
        x = xs_ref[pl.ds(r0, B), :] * ss_ref[pl.ds(r0, B), :]
        gu = jnp.dot(x, wgu_ref[0], preferred_element_type=jnp.float32)
        g = gu[:, :F]
        u = gu[:, F:]
        act = u * (g * jax.nn.sigmoid(g))
        r = jnp.dot(act, wd_ref[0], preferred_element_type=jnp.float32)
        rows = r0 + lax.broadcasted_iota(jnp.int32, (B, 1), 0)
        keep = rows >= start
        out_ref[pl.ds(r0, B), :] = jnp.where(
            keep, r, out_ref[pl.ds(r0, B), :])

    @pl.when(e == E)
    def _shared():
        x = xs_ref[pl.ds(r0, B), :]
        g = lax.dot_general(x, wg_ref[...], dn,
                            preferred_element_type=jnp.float32)
        u = lax.dot_general(x, wu_ref[...], dn,
                            preferred_element_type=jnp.float32)
        act = u * (g * jax.nn.sigmoid(g))
        sh = lax.dot_general(act, wdw_ref[...], dn,
                             preferred_element_type=jnp.float32)
        out_ref[pl.ds(r0, B), :] = out_ref[pl.ds(r0, B), :] + sh


def _moe(off, xs, ss, gate_up_proj, down_proj, gate_w, up_w, down_w):
    return pl.pallas_call(
        _moe_body,
        grid=(E + 1, NB),
        in_specs=[
            pl.BlockSpec(memory_space=pltpu.SMEM),
            pl.BlockSpec((T, H), lambda e, b: (0, 0)),
            pl.BlockSpec((T, 1), lambda e, b: (0, 0)),
            pl.BlockSpec((1, H, 2 * F), lambda e, b: (jnp.minimum(e, E - 1), 0, 0)),
            pl.BlockSpec((1, F, H), lambda e, b: (jnp.minimum(e, E - 1), 0, 0)),
            pl.BlockSpec((F, H), lambda e, b: (0, 0)),
            pl.BlockSpec((F, H), lambda e, b: (0, 0)),
            pl.BlockSpec((H, F), lambda e, b: (0, 0)),
        ],
        out_specs=pl.BlockSpec((T, H), lambda e, b: (0, 0)),
        out_shape=jax.ShapeDtypeStruct((T, H), jnp.float32),
        compiler_params=pltpu.CompilerParams(
            dimension_semantics=("arbitrary", "arbitrary")),
    )(off, xs, ss, gate_up_proj, down_proj, gate_w, up_w, down_w)


def _combine_body(osort_hbm, pos_hbm, out_hbm, idx_v, rows_v, sem):
    wid = lax.axis_index("s") * NC + lax.axis_index("c")
    base = wid * BPW
    pltpu.sync_copy(pos_hbm.at[pl.ds(base, BPW)], idx_v)
    pltpu.async_copy(osort_hbm.at[idx_v], rows_v, sem).wait()
    pltpu.sync_copy(rows_v, out_hbm.at[pl.ds(base, BPW)])


def _combine(osort, pos):
    mesh = plsc.VectorSubcoreMesh(core_axis_name="c", subcore_axis_name="s",
                                  num_cores=NC, num_subcores=NS)
    return pl.kernel(
        _combine_body,
        out_type=jax.ShapeDtypeStruct((T, H), jnp.float32),
        mesh=mesh,
        scratch_types=[
            pltpu.VMEM((BPW,), jnp.int32),
            pltpu.VMEM((BPW, H), jnp.float32),
            pltpu.SemaphoreType.DMA,
        ],
    )(osort, pos)


def kernel(hidden_states, router_w, gate_up_proj, down_proj,
           gate_w, up_w, down_w):
    orig_shape = hidden_states.shape
    hs = hidden_states.reshape(-1, H)
    pos2d, score2d, base2d = _router(hs, router_w)
    pos = pos2d.reshape(T)
    score = score2d.reshape(T)
    off = jnp.concatenate(
        [base2d.reshape(E), jnp.full((E,), T, jnp.int32)])
    xs, s_sorted = _disperse(hs, score, pos)
    osort = _moe(off, xs, s_sorted.reshape(T, 1),
                 gate_up_proj, down_proj, gate_w, up_w, down_w)
    out = _combine(osort, pos)
    return out.reshape(orig_shape)
